# baseline (speedup 1.0000x reference)
import numpy as np
import jax
import jax.numpy as jnp
from jax.experimental import pallas as pl
from jax.experimental.pallas import tpu as pltpu

_UP_CFG = [
    (8, 8, 15, 4, 4),
    (15, 16, 31, 4, 8),
    (31, 32, 63, 8, 16),
    (63, 64, 126, 16, 32),
]
_KP = [(5, 2), (5, 1), (5, 1), (4, 1)]
_BC = 2


def _up_select(K, p):
    S = np.zeros((3, 3, 2, 2, K, K), np.float32)
    for a in (0, 1):
        pia, ca = (a + p) % 2, (a + p) // 2
        for bb in (0, 1):
            pib, cb = (bb + p) % 2, (bb + p) // 2
            for t in range((K - pia + 1) // 2):
                for r in range((K - pib + 1) // 2):
                    S[1 + ca - t, 1 + cb - r, a, bb, pia + 2 * t,
                      pib + 2 * r] = 1.0
    return S


def _prep_up_weight(w, b, K, p):
    Cin, Cout = w.shape[0], w.shape[1]
    S = jnp.asarray(_up_select(K, p))
    Wu = jnp.einsum("dwabkl,iokl->dwiabo", S, w)
    return Wu.reshape(9 * Cin, 4 * Cout), jnp.tile(b, 4).reshape(1, 4 * Cout)


def _decoder_body(y_ref, w0, b0, w1, b1, w2, b2, w3, b3, w4, b4, o_ref):
    f32 = jnp.float32
    Bc = _BC

    def up_gemm(act, w_ref, b_ref, Hi, HU):
        HP = HU + 2
        xp = jnp.pad(act, ((0, 0), (1, HP - 1 - Hi), (1, HP - 1 - Hi),
                           (0, 0)))
        patch = jnp.concatenate(
            [xp[:, dh:dh + HU, dw:dw + HU, :]
             for dh in range(3) for dw in range(3)], axis=-1)
        patch = patch.reshape(Bc * HU * HU, patch.shape[-1])
        r = jnp.dot(patch, w_ref[...], preferred_element_type=f32)
        return jnp.maximum(r + b_ref[...], 0.0)

    def interleave(r, HU, Ho, Cout):
        r = r.reshape(Bc, HU, HU, 2, 2, Cout)
        ra0 = r[:, :, :, 0].reshape(Bc, HU, 2 * HU, Cout)
        ra1 = r[:, :, :, 1].reshape(Bc, HU, 2 * HU, Cout)
        full = jnp.stack([ra0, ra1], axis=2).reshape(Bc, 2 * HU, 2 * HU, Cout)
        return full[:, :Ho, :Ho, :]

    act = y_ref[...]
    for i, (Hi, HU, Ho, _Cin, Cout) in enumerate(_UP_CFG[:3]):
        w_ref, b_ref = (w0, b0, w1, b1, w2, b2)[2 * i:2 * i + 2]
        act = interleave(up_gemm(act, w_ref, b_ref, Hi, HU), HU, Ho, Cout)

    r3 = up_gemm(act, w3, b3, 63, 64).reshape(Bc, 64, 64, 128)
    r3m = jnp.pad(r3[:, :63, :63, :], ((0, 0), (1, 1), (1, 1), (0, 0)))

    w4f = w4[...].reshape(288, 3)
    for g in (0, 1):
        for dd in (0, 1):
            pieces = []
            for dh in range(3):
                al = (g + dh) % 2
                mo = (g + dh - 2 - al) // 2
                for dw in range(3):
                    be = (dd + dw) % 2
                    wo = (dd + dw - 2 - be) // 2
                    lane = (2 * al + be) * 32
                    pieces.append(
                        r3m[:, 1 + mo:65 + mo, 1 + wo:65 + wo,
                            lane:lane + 32])
            patch = jnp.concatenate(pieces, axis=-1).reshape(Bc * 4096, 288)
            q = jnp.dot(patch, w4f, preferred_element_type=f32)
            q = jnp.maximum(q + b4[...], 0.0).reshape(Bc, 4096, 3)
            qT = jnp.transpose(q, (0, 2, 1)).reshape(Bc, 3, 64, 64)
            o_ref[:, :, g, dd] = qT


def kernel(x, lin_w, lin_b, conv0_w, conv0_b, conv1_w, conv1_b, conv2_w,
           conv2_b, conv3_w, conv3_b, conv4_w, conv4_b):
    B = x.shape[0]
    y = jnp.maximum(x @ lin_w.T + lin_b, 0.0)
    act0 = y.reshape(B, 4, 8, 8).transpose(0, 2, 3, 1)

    convs = [(conv0_w, conv0_b), (conv1_w, conv1_b), (conv2_w, conv2_b),
             (conv3_w, conv3_b)]
    args = [act0]
    for (w, b), (K, p) in zip(convs, _KP):
        Wu, bu = _prep_up_weight(w, b, K, p)
        args += [Wu, bu]
    W4 = jnp.flip(conv4_w, (2, 3)).transpose(2, 3, 0, 1).reshape(3, 96, 3)
    args += [W4, conv4_b.reshape(1, 3)]

    const = lambda shape: pl.BlockSpec(shape, lambda b: (0,) * len(shape))
    in_specs = [pl.BlockSpec((_BC, 8, 8, 4), lambda b: (b, 0, 0, 0))]
    for a in args[1:]:
        in_specs.append(const(a.shape))

    out = pl.pallas_call(
        _decoder_body,
        grid=(B // _BC,),
        in_specs=in_specs,
        out_specs=pl.BlockSpec((_BC, 3, 2, 2, 64, 64),
                               lambda b: (b, 0, 0, 0, 0, 0)),
        out_shape=jax.ShapeDtypeStruct((B, 3, 2, 2, 64, 64), jnp.float32),
        compiler_params=pltpu.CompilerParams(
            dimension_semantics=("parallel",)),
    )(*args)
    out = out.transpose(0, 1, 4, 2, 5, 3).reshape(B, 3, 128, 128)
    return out


# speedup vs baseline: 1.3394x; 1.3394x over previous
import numpy as np
import jax
import jax.numpy as jnp
from jax.experimental import pallas as pl
from jax.experimental.pallas import tpu as pltpu

_UP_CFG = [
    (8, 8, 15, 4, 4),
    (15, 16, 31, 4, 8),
    (31, 32, 63, 8, 16),
    (63, 64, 126, 16, 32),
]
_KP = [(5, 2), (5, 1), (5, 1), (4, 1)]
_BC = 2


def _up_select(K, p):
    S = np.zeros((3, 3, 2, 2, K, K), np.float32)
    for a in (0, 1):
        pia, ca = (a + p) % 2, (a + p) // 2
        for bb in (0, 1):
            pib, cb = (bb + p) % 2, (bb + p) // 2
            for t in range((K - pia + 1) // 2):
                for r in range((K - pib + 1) // 2):
                    S[1 + ca - t, 1 + cb - r, a, bb, pia + 2 * t,
                      pib + 2 * r] = 1.0
    return S


def _prep_up_weight(w, b, K, p):
    Cin, Cout = w.shape[0], w.shape[1]
    S = jnp.asarray(_up_select(K, p))
    Wu = jnp.einsum("dwabkl,iokl->dwiabo", S, w)
    return Wu.reshape(9 * Cin, 4 * Cout), jnp.tile(b, 4).reshape(1, 4 * Cout)


def _final_select():
    T = np.zeros((3, 3, 2, 2, 2, 2, 2, 2), np.float32)
    for g in (0, 1):
        for d in (0, 1):
            for dh in range(3):
                al = (g + dh) % 2
                mo = (g + dh - 2 - al) // 2 + 1
                for dw in range(3):
                    be = (d + dw) % 2
                    wo = (d + dw - 2 - be) // 2 + 1
                    T[dh, dw, mo, wo, al, be, g, d] = 1.0
    return T


def _prep_final_weight(w, b):
    A = jnp.flip(w, (2, 3)).transpose(2, 3, 0, 1)
    T = jnp.asarray(_final_select())
    W4q = jnp.einsum("hwmnabgd,hwio->mnabigdo", T, A)
    return W4q.reshape(512, 12), jnp.tile(b, 4).reshape(12, 1)


def _decoder_body(y_ref, w0, b0, w1, b1, w2, b2, w3, b3, w4, b4, o_ref):
    f32 = jnp.float32
    Bc = _BC

    def up_gemm(act, w_ref, b_ref, Hi, HU):
        HP = HU + 2
        xp = jnp.pad(act, ((0, 0), (1, HP - 1 - Hi), (1, HP - 1 - Hi),
                           (0, 0)))
        patch = jnp.concatenate(
            [xp[:, dh:dh + HU, dw:dw + HU, :]
             for dh in range(3) for dw in range(3)], axis=-1)
        patch = patch.reshape(Bc * HU * HU, patch.shape[-1])
        r = jnp.dot(patch, w_ref[...], preferred_element_type=f32)
        return jnp.maximum(r + b_ref[...], 0.0)

    def interleave(r, HU, Ho, Cout):
        r = r.reshape(Bc, HU, HU, 2, 2, Cout)
        ra0 = r[:, :, :, 0].reshape(Bc, HU, 2 * HU, Cout)
        ra1 = r[:, :, :, 1].reshape(Bc, HU, 2 * HU, Cout)
        full = jnp.stack([ra0, ra1], axis=2).reshape(Bc, 2 * HU, 2 * HU, Cout)
        return full[:, :Ho, :Ho, :]

    act = y_ref[...]
    for i, (Hi, HU, Ho, _Cin, Cout) in enumerate(_UP_CFG[:3]):
        w_ref, b_ref = (w0, b0, w1, b1, w2, b2)[2 * i:2 * i + 2]
        act = interleave(up_gemm(act, w_ref, b_ref, Hi, HU), HU, Ho, Cout)

    r3 = up_gemm(act.astype(jnp.bfloat16), w3, b3, 63, 64)
    r3 = r3.astype(jnp.bfloat16).reshape(Bc, 64, 64, 128)
    r3m = jnp.pad(r3[:, :63, :63, :], ((0, 0), (1, 1), (1, 1), (0, 0)))

    patch = jnp.concatenate(
        [r3m[:, mo:64 + mo, wo:64 + wo, :]
         for mo in (0, 1) for wo in (0, 1)], axis=-1)
    patch = patch.reshape(Bc * 4096, 512)
    z = jnp.dot(patch, w4[...], preferred_element_type=f32)
    zT = jnp.maximum(z.T + b4[...], 0.0)
    for g in (0, 1):
        for dd in (0, 1):
            rows = (2 * g + dd) * 3
            q = zT[rows:rows + 3].reshape(3, Bc, 64, 64)
            o_ref[:, :, g, dd] = jnp.transpose(q, (1, 0, 2, 3))


def kernel(x, lin_w, lin_b, conv0_w, conv0_b, conv1_w, conv1_b, conv2_w,
           conv2_b, conv3_w, conv3_b, conv4_w, conv4_b):
    B = x.shape[0]
    y = jnp.maximum(x @ lin_w.T + lin_b, 0.0)
    act0 = y.reshape(B, 4, 8, 8).transpose(0, 2, 3, 1)

    convs = [(conv0_w, conv0_b), (conv1_w, conv1_b), (conv2_w, conv2_b),
             (conv3_w, conv3_b)]
    args = [act0]
    for i, ((w, b), (K, p)) in enumerate(zip(convs, _KP)):
        Wu, bu = _prep_up_weight(w, b, K, p)
        if i == 3:
            Wu = Wu.astype(jnp.bfloat16)
        args += [Wu, bu]
    W4q, b4q = _prep_final_weight(conv4_w, conv4_b)
    args += [W4q.astype(jnp.bfloat16), b4q]

    const = lambda shape: pl.BlockSpec(shape, lambda b: (0,) * len(shape))
    in_specs = [pl.BlockSpec((_BC, 8, 8, 4), lambda b: (b, 0, 0, 0))]
    for a in args[1:]:
        in_specs.append(const(a.shape))

    out = pl.pallas_call(
        _decoder_body,
        grid=(B // _BC,),
        in_specs=in_specs,
        out_specs=pl.BlockSpec((_BC, 3, 2, 2, 64, 64),
                               lambda b: (b, 0, 0, 0, 0, 0)),
        out_shape=jax.ShapeDtypeStruct((B, 3, 2, 2, 64, 64), jnp.float32),
        compiler_params=pltpu.CompilerParams(
            dimension_semantics=("parallel",)),
    )(*args)
    out = out.transpose(0, 1, 4, 2, 5, 3).reshape(B, 3, 128, 128)
    return out


# speedup vs baseline: 1.3577x; 1.0136x over previous
import numpy as np
import jax
import jax.numpy as jnp
from jax.experimental import pallas as pl
from jax.experimental.pallas import tpu as pltpu

_UP_CFG = [
    (8, 8, 15, 4, 4),
    (15, 16, 31, 4, 8),
    (31, 32, 63, 8, 16),
    (63, 64, 126, 16, 32),
]
_KP = [(5, 2), (5, 1), (5, 1), (4, 1)]
_BC = 4


def _up_select(K, p):
    S = np.zeros((3, 3, 2, 2, K, K), np.float32)
    for a in (0, 1):
        pia, ca = (a + p) % 2, (a + p) // 2
        for bb in (0, 1):
            pib, cb = (bb + p) % 2, (bb + p) // 2
            for t in range((K - pia + 1) // 2):
                for r in range((K - pib + 1) // 2):
                    S[1 + ca - t, 1 + cb - r, a, bb, pia + 2 * t,
                      pib + 2 * r] = 1.0
    return S


def _prep_up_weight(w, b, K, p):
    Cin, Cout = w.shape[0], w.shape[1]
    S = jnp.asarray(_up_select(K, p))
    Wu = jnp.einsum("dwabkl,iokl->dwiabo", S, w)
    return Wu.reshape(9 * Cin, 4 * Cout), jnp.tile(b, 4).reshape(1, 4 * Cout)


def _final_select():
    T = np.zeros((3, 3, 2, 2, 2, 2, 2, 2), np.float32)
    for g in (0, 1):
        for d in (0, 1):
            for dh in range(3):
                al = (g + dh) % 2
                mo = (g + dh - 2 - al) // 2 + 1
                for dw in range(3):
                    be = (d + dw) % 2
                    wo = (d + dw - 2 - be) // 2 + 1
                    T[dh, dw, mo, wo, al, be, g, d] = 1.0
    return T


def _prep_final_weight(w, b):
    A = jnp.flip(w, (2, 3)).transpose(2, 3, 0, 1)
    T = jnp.asarray(_final_select())
    W4q = jnp.einsum("hwmnabgd,hwio->mnabigdo", T, A)
    return W4q.reshape(512, 12), jnp.tile(b, 4).reshape(12, 1)


def _decoder_body(y_ref, w0, b0, w1, b1, w2, b2, w3, b3, w4, b4, o_ref):
    f32 = jnp.float32
    Bc = _BC

    def up_gemm(act, w_ref, b_ref, Hi, HU):
        HP = HU + 2
        xp = jnp.pad(act, ((0, 0), (1, HP - 1 - Hi), (1, HP - 1 - Hi),
                           (0, 0)))
        patch = jnp.concatenate(
            [xp[:, dh:dh + HU, dw:dw + HU, :]
             for dh in range(3) for dw in range(3)], axis=-1)
        patch = patch.reshape(Bc * HU * HU, patch.shape[-1])
        r = jnp.dot(patch, w_ref[...], preferred_element_type=f32)
        return jnp.maximum(r + b_ref[...], 0.0)

    def interleave(r, HU, Ho, Cout):
        r = r.reshape(Bc, HU, HU, 2, 2, Cout)
        ra0 = r[:, :, :, 0].reshape(Bc, HU, 2 * HU, Cout)
        ra1 = r[:, :, :, 1].reshape(Bc, HU, 2 * HU, Cout)
        full = jnp.stack([ra0, ra1], axis=2).reshape(Bc, 2 * HU, 2 * HU, Cout)
        return full[:, :Ho, :Ho, :]

    act = y_ref[...]
    for i, (Hi, HU, Ho, _Cin, Cout) in enumerate(_UP_CFG[:3]):
        w_ref, b_ref = (w0, b0, w1, b1, w2, b2)[2 * i:2 * i + 2]
        act = interleave(up_gemm(act, w_ref, b_ref, Hi, HU), HU, Ho, Cout)

    r3 = up_gemm(act.astype(jnp.bfloat16), w3, b3, 63, 64)
    r3 = r3.astype(jnp.bfloat16).reshape(Bc, 64, 64, 128)
    r3m = jnp.pad(r3[:, :63, :63, :], ((0, 0), (1, 1), (1, 1), (0, 0)))

    patch = jnp.concatenate(
        [r3m[:, mo:64 + mo, wo:64 + wo, :]
         for mo in (0, 1) for wo in (0, 1)], axis=-1)
    patch = patch.reshape(Bc * 4096, 512)
    z = jnp.dot(patch, w4[...], preferred_element_type=f32)
    zT = jnp.maximum(z.T + b4[...], 0.0)
    for g in (0, 1):
        for dd in (0, 1):
            rows = (2 * g + dd) * 3
            q = zT[rows:rows + 3].reshape(3, Bc, 64, 64)
            o_ref[:, :, g, dd] = jnp.transpose(q, (1, 0, 2, 3))


def kernel(x, lin_w, lin_b, conv0_w, conv0_b, conv1_w, conv1_b, conv2_w,
           conv2_b, conv3_w, conv3_b, conv4_w, conv4_b):
    B = x.shape[0]
    y = jnp.maximum(x @ lin_w.T + lin_b, 0.0)
    act0 = y.reshape(B, 4, 8, 8).transpose(0, 2, 3, 1)

    convs = [(conv0_w, conv0_b), (conv1_w, conv1_b), (conv2_w, conv2_b),
             (conv3_w, conv3_b)]
    args = [act0]
    for i, ((w, b), (K, p)) in enumerate(zip(convs, _KP)):
        Wu, bu = _prep_up_weight(w, b, K, p)
        if i == 3:
            Wu = Wu.astype(jnp.bfloat16)
        args += [Wu, bu]
    W4q, b4q = _prep_final_weight(conv4_w, conv4_b)
    args += [W4q.astype(jnp.bfloat16), b4q]

    const = lambda shape: pl.BlockSpec(shape, lambda b: (0,) * len(shape))
    in_specs = [pl.BlockSpec((_BC, 8, 8, 4), lambda b: (b, 0, 0, 0))]
    for a in args[1:]:
        in_specs.append(const(a.shape))

    out = pl.pallas_call(
        _decoder_body,
        grid=(B // _BC,),
        in_specs=in_specs,
        out_specs=pl.BlockSpec((_BC, 3, 2, 2, 64, 64),
                               lambda b: (b, 0, 0, 0, 0, 0)),
        out_shape=jax.ShapeDtypeStruct((B, 3, 2, 2, 64, 64), jnp.float32),
        compiler_params=pltpu.CompilerParams(
            dimension_semantics=("parallel",)),
    )(*args)
    out = out.transpose(0, 1, 4, 2, 5, 3).reshape(B, 3, 128, 128)
    return out


# speedup vs baseline: 1.4404x; 1.0610x over previous
import numpy as np
import jax
import jax.numpy as jnp
from jax.experimental import pallas as pl
from jax.experimental.pallas import tpu as pltpu

_UP_CFG = [
    (8, 8, 15, 4, 4),
    (15, 16, 31, 4, 8),
    (31, 32, 63, 8, 16),
    (63, 64, 126, 16, 32),
]
_KP = [(5, 2), (5, 1), (5, 1), (4, 1)]
_BC = 4


def _up_select(K, p):
    S = np.zeros((3, 3, 2, 2, K, K), np.float32)
    for a in (0, 1):
        pia, ca = (a + p) % 2, (a + p) // 2
        for bb in (0, 1):
            pib, cb = (bb + p) % 2, (bb + p) // 2
            for t in range((K - pia + 1) // 2):
                for r in range((K - pib + 1) // 2):
                    S[1 + ca - t, 1 + cb - r, a, bb, pia + 2 * t,
                      pib + 2 * r] = 1.0
    return S


def _prep_up_weight(w, b, K, p):
    Cin, Cout = w.shape[0], w.shape[1]
    S = jnp.asarray(_up_select(K, p))
    Wu = jnp.einsum("dwabkl,iokl->dwiabo", S, w)
    return Wu.reshape(9 * Cin, 4 * Cout), jnp.tile(b, 4).reshape(1, 4 * Cout)


def _final_select():
    T = np.zeros((3, 3, 2, 2, 2, 2, 2, 2), np.float32)
    for g in (0, 1):
        for d in (0, 1):
            for dh in range(3):
                al = (g + dh) % 2
                mo = (g + dh - 2 - al) // 2 + 1
                for dw in range(3):
                    be = (d + dw) % 2
                    wo = (d + dw - 2 - be) // 2 + 1
                    T[dh, dw, mo, wo, al, be, g, d] = 1.0
    return T


def _prep_final_weight(w, b):
    A = jnp.flip(w, (2, 3)).transpose(2, 3, 0, 1)
    T = jnp.asarray(_final_select())
    W4q = jnp.einsum("hwmnabgd,hwio->mnabigdo", T, A)
    return W4q.reshape(512, 12), jnp.tile(b, 4).reshape(12, 1)


def _decoder_body(y_ref, w0, b0, w1, b1, w2, b2, w3, b3, w4, b4, o_ref):
    f32 = jnp.float32
    Bc = _BC

    def up_gemm(act, w_ref, b_ref, Hi, HU):
        HP = HU + 2
        xp = jnp.pad(act, ((0, 0), (1, HP - 1 - Hi), (1, HP - 1 - Hi),
                           (0, 0)))
        patch = jnp.concatenate(
            [xp[:, dh:dh + HU, dw:dw + HU, :]
             for dh in range(3) for dw in range(3)], axis=-1)
        patch = patch.reshape(Bc * HU * HU, patch.shape[-1])
        r = jnp.dot(patch, w_ref[...], preferred_element_type=f32)
        return jnp.maximum(r + b_ref[...], 0.0).astype(jnp.bfloat16)

    def interleave(r, HU, Ho, Cout):
        r = r.reshape(Bc, HU, HU, 2, 2, Cout)
        ra0 = r[:, :, :, 0].reshape(Bc, HU, 2 * HU, Cout)
        ra1 = r[:, :, :, 1].reshape(Bc, HU, 2 * HU, Cout)
        full = jnp.stack([ra0, ra1], axis=2).reshape(Bc, 2 * HU, 2 * HU, Cout)
        return full[:, :Ho, :Ho, :]

    act = y_ref[...].astype(jnp.bfloat16)
    for i, (Hi, HU, Ho, _Cin, Cout) in enumerate(_UP_CFG[:3]):
        w_ref, b_ref = (w0, b0, w1, b1, w2, b2)[2 * i:2 * i + 2]
        act = interleave(up_gemm(act, w_ref, b_ref, Hi, HU), HU, Ho, Cout)

    r3 = up_gemm(act, w3, b3, 63, 64).reshape(Bc, 64, 64, 128)
    r3m = jnp.pad(r3[:, :63, :63, :], ((0, 0), (1, 1), (1, 1), (0, 0)))

    patch = jnp.concatenate(
        [r3m[:, mo:64 + mo, wo:64 + wo, :]
         for mo in (0, 1) for wo in (0, 1)], axis=-1)
    patch = patch.reshape(Bc * 4096, 512)
    z = jnp.dot(patch, w4[...], preferred_element_type=f32)
    zT = jnp.maximum(z.T + b4[...], 0.0)
    for g in (0, 1):
        for dd in (0, 1):
            rows = (2 * g + dd) * 3
            q = zT[rows:rows + 3].reshape(3, Bc, 64, 64)
            o_ref[:, :, g, dd] = jnp.transpose(q, (1, 0, 2, 3))


def kernel(x, lin_w, lin_b, conv0_w, conv0_b, conv1_w, conv1_b, conv2_w,
           conv2_b, conv3_w, conv3_b, conv4_w, conv4_b):
    B = x.shape[0]
    y = jnp.maximum(x @ lin_w.T + lin_b, 0.0)
    act0 = y.reshape(B, 4, 8, 8).transpose(0, 2, 3, 1)

    convs = [(conv0_w, conv0_b), (conv1_w, conv1_b), (conv2_w, conv2_b),
             (conv3_w, conv3_b)]
    args = [act0]
    for (w, b), (K, p) in zip(convs, _KP):
        Wu, bu = _prep_up_weight(w, b, K, p)
        args += [Wu.astype(jnp.bfloat16), bu]
    W4q, b4q = _prep_final_weight(conv4_w, conv4_b)
    args += [W4q.astype(jnp.bfloat16), b4q]

    const = lambda shape: pl.BlockSpec(shape, lambda b: (0,) * len(shape))
    in_specs = [pl.BlockSpec((_BC, 8, 8, 4), lambda b: (b, 0, 0, 0))]
    for a in args[1:]:
        in_specs.append(const(a.shape))

    out = pl.pallas_call(
        _decoder_body,
        grid=(B // _BC,),
        in_specs=in_specs,
        out_specs=pl.BlockSpec((_BC, 3, 2, 2, 64, 64),
                               lambda b: (b, 0, 0, 0, 0, 0)),
        out_shape=jax.ShapeDtypeStruct((B, 3, 2, 2, 64, 64), jnp.float32),
        compiler_params=pltpu.CompilerParams(
            dimension_semantics=("parallel",)),
    )(*args)
    out = out.transpose(0, 1, 4, 2, 5, 3).reshape(B, 3, 128, 128)
    return out
